# baseline (speedup 1.0000x reference)
import jax
import jax.numpy as jnp
from jax import lax
from jax.experimental import pallas as pl
from jax.experimental.pallas import tpu as pltpu
from jax.experimental.pallas import tpu_sc as plsc

_N_MAX = 20
_REG_WEIGHT = 0.01
_ROWS, _COLS = 128, 32768
_ROW_BLOCK = 16
_NSTEPS = _ROWS // _ROW_BLOCK

_NW = 16
_CHUNK = _COLS // _NW
_NGROUP = _CHUNK // 256
_NSLOT = 32
_BIG = 3.0e38


def _l1_body(t_ref, p_ref, l1_ref):
    step = pl.program_id(0)
    bsum = jnp.sum(jnp.abs(t_ref[...] - p_ref[...]))

    @pl.when(step == 0)
    def _init():
        l1_ref[...] = jnp.reshape(bsum, (1, 1))

    @pl.when(step != 0)
    def _acc():
        l1_ref[...] += jnp.reshape(bsum, (1, 1))

    @pl.when(step == _NSTEPS - 1)
    def _fin():
        l1_ref[...] = l1_ref[...] / float(_ROWS * _COLS)


def _l1_mean(target, pred):
    out = pl.pallas_call(
        _l1_body,
        grid=(_NSTEPS,),
        in_specs=[
            pl.BlockSpec((_ROW_BLOCK, _COLS), lambda i: (i, 0)),
            pl.BlockSpec((_ROW_BLOCK, _COLS), lambda i: (i, 0)),
        ],
        out_specs=pl.BlockSpec((1, 1), lambda i: (0, 0)),
        out_shape=jax.ShapeDtypeStruct((1, 1), jnp.float32),
        compiler_params=pltpu.CompilerParams(
            dimension_semantics=("arbitrary",),
        ),
    )(target, pred)
    return out[0, 0]


def _iota16():
    return lax.iota(jnp.int32, 16)


def _splat_f(x):
    return jnp.full((16,), x, dtype=jnp.float32)


def _splat_i(x):
    return jnp.full((16,), x, dtype=jnp.int32)


def _sqrt16(a):
    ai = plsc.bitcast(a, jnp.int32)
    x = plsc.bitcast((ai >> _splat_i(1)) + _splat_i(0x1FBD1DF5), jnp.float32)
    for _ in range(3):
        x = _splat_f(0.5) * (x + a / x)
    return x


def _argmax_pos(vec, m):
    return jnp.min(jnp.where(vec == m, _iota16(), _splat_i(9999)))


def _reg_body(lat_ref, r_ref, out_ref,
              chunk_v, rchunk_v, mval_v, mx_v, my_v, mz_v,
              xs_v, ys_v, zs_v,
              vstage_v, xstage_v, ystage_v, zstage_v, ostage_v,
              shval, shx, shy, shz):
    cid = lax.axis_index("c")
    sid = lax.axis_index("s")
    lane = _iota16()
    lane0 = lane == 0

    @pl.when(cid == 0)
    def _local():
        base = sid * _CHUNK
        pltpu.sync_copy(lat_ref.at[pl.ds(base, _CHUNK)], chunk_v)
        for c in range(3):
            pltpu.sync_copy(r_ref.at[pl.ds(c * _COLS + base, _CHUNK)],
                            rchunk_v.at[pl.ds(c * _CHUNK, _CHUNK)])

        g_list = []
        for g in range(_NGROUP):
            acc = chunk_v[pl.ds(g * 256, 16)]
            for j in range(1, 16):
                acc = jnp.maximum(acc, chunk_v[pl.ds(g * 256 + j * 16, 16)])
            g_list.append(acc)

        def _round(k, carry):
            pv0, pv1, pi0, pi1, *gs = carry
            mall = gs[0]
            for g in range(1, _NGROUP):
                mall = jnp.maximum(mall, gs[g])
            m = jnp.max(mall)
            g_star = jnp.int32(9999)
            for g in range(_NGROUP):
                hit = jnp.max(jnp.where(gs[g] == m, _splat_i(1), _splat_i(0)))
                g_star = jnp.minimum(
                    g_star, jnp.where(hit > 0, jnp.int32(g), jnp.int32(9999)))
            gsel = _splat_f(-_BIG)
            for g in range(_NGROUP):
                gsel = jnp.where(_splat_i(g_star) == g, gs[g], gsel)
            l_star = _argmax_pos(gsel, m)
            colidx = g_star * 256 + lane * 16 + l_star
            col = plsc.load_gather(chunk_v, [colidx])
            r_off = jnp.min(jnp.where(col == m, lane, _splat_i(9999)))
            flat = g_star * 256 + r_off * 16 + l_star
            plsc.store_scatter(chunk_v, [_splat_i(flat)], _splat_f(-_BIG),
                               mask=lane0)
            newg = _splat_f(-_BIG)
            for j in range(16):
                rowv = plsc.load_gather(chunk_v,
                                        [g_star * 256 + j * 16 + lane])
                newg = jnp.maximum(newg, rowv)
            new_gs = [jnp.where(_splat_i(g_star) == g, newg, gs[g])
                      for g in range(_NGROUP)]
            pv0 = jnp.where(lane == k, _splat_f(m), pv0)
            pv1 = jnp.where(lane == k - 16, _splat_f(m), pv1)
            pi0 = jnp.where(lane == k, _splat_i(flat), pi0)
            pi1 = jnp.where(lane == k - 16, _splat_i(flat), pi1)
            return (pv0, pv1, pi0, pi1, *new_gs)

        init = (_splat_f(-_BIG), _splat_f(-_BIG),
                _splat_i(0), _splat_i(0), *g_list)
        pv0, pv1, pi0, pi1, *_ = lax.fori_loop(0, _N_MAX, _round, init)

        vstage_v[pl.ds(0, 16)] = pv0
        vstage_v[pl.ds(16, 16)] = pv1
        xstage_v[pl.ds(0, 16)] = plsc.load_gather(rchunk_v, [pi0])
        xstage_v[pl.ds(16, 16)] = plsc.load_gather(rchunk_v, [pi1])
        ystage_v[pl.ds(0, 16)] = plsc.load_gather(rchunk_v, [pi0 + _CHUNK])
        ystage_v[pl.ds(16, 16)] = plsc.load_gather(rchunk_v, [pi1 + _CHUNK])
        zstage_v[pl.ds(0, 16)] = plsc.load_gather(rchunk_v, [pi0 + 2 * _CHUNK])
        zstage_v[pl.ds(16, 16)] = plsc.load_gather(rchunk_v, [pi1 + 2 * _CHUNK])
        pltpu.sync_copy(vstage_v, shval.at[pl.ds(sid * _NSLOT, _NSLOT)])
        pltpu.sync_copy(xstage_v, shx.at[pl.ds(sid * _NSLOT, _NSLOT)])
        pltpu.sync_copy(ystage_v, shy.at[pl.ds(sid * _NSLOT, _NSLOT)])
        pltpu.sync_copy(zstage_v, shz.at[pl.ds(sid * _NSLOT, _NSLOT)])

    plsc.subcore_barrier()

    @pl.when((cid == 0) & (sid == 0))
    def _merge():
        pltpu.sync_copy(shval, mval_v)
        pltpu.sync_copy(shx, mx_v)
        pltpu.sync_copy(shy, my_v)
        pltpu.sync_copy(shz, mz_v)

        nrows = _NW * _NSLOT // 16

        def _round(k, carry):
            wx0, wx1, wy0, wy1, wz0, wz1 = carry
            mall = mval_v[pl.ds(0, 16)]
            for j in range(1, nrows):
                mall = jnp.maximum(mall, mval_v[pl.ds(j * 16, 16)])
            m = jnp.max(mall)
            l_star = _argmax_pos(mall, m)
            r_star = jnp.int32(9999)
            for h in range(nrows // 16):
                colidx = (lane + h * 16) * 16 + l_star
                col = plsc.load_gather(mval_v, [colidx])
                cand = jnp.min(jnp.where(col == m, lane + h * 16,
                                         _splat_i(9999)))
                r_star = jnp.minimum(r_star, cand)
            flat = r_star * 16 + l_star
            plsc.store_scatter(mval_v, [_splat_i(flat)], _splat_f(-_BIG),
                               mask=lane0)
            fl = _splat_i(flat)
            gx = plsc.load_gather(mx_v, [fl])
            gy = plsc.load_gather(my_v, [fl])
            gz = plsc.load_gather(mz_v, [fl])
            wx0 = jnp.where(lane == k, gx, wx0)
            wx1 = jnp.where(lane == k - 16, gx, wx1)
            wy0 = jnp.where(lane == k, gy, wy0)
            wy1 = jnp.where(lane == k - 16, gy, wy1)
            wz0 = jnp.where(lane == k, gz, wz0)
            wz1 = jnp.where(lane == k - 16, gz, wz1)
            return (wx0, wx1, wy0, wy1, wz0, wz1)

        zero = _splat_f(0.0)
        wx0, wx1, wy0, wy1, wz0, wz1 = lax.fori_loop(
            0, _N_MAX, _round, (zero, zero, zero, zero, zero, zero))

        xs_v[pl.ds(0, 16)] = wx0
        xs_v[pl.ds(16, 16)] = wx1
        ys_v[pl.ds(0, 16)] = wy0
        ys_v[pl.ds(16, 16)] = wy1
        zs_v[pl.ds(0, 16)] = wz0
        zs_v[pl.ds(16, 16)] = wz1

        def _pair(i, carry):
            s1a, s1b, s2a, s2b = carry
            xi = plsc.load_gather(xs_v, [_splat_i(i)])
            yi = plsc.load_gather(ys_v, [_splat_i(i)])
            zi = plsc.load_gather(zs_v, [_splat_i(i)])
            dxa = wx0 - xi
            dya = wy0 - yi
            dza = wz0 - zi
            dxb = wx1 - xi
            dyb = wy1 - yi
            dzb = wz1 - zi
            d2a = dxa * dxa + dya * dya + dza * dza
            d2b = dxb * dxb + dyb * dyb + dzb * dzb
            maska = (lane > i) & (lane < _N_MAX)
            maskb = ((lane + 16) > i) & ((lane + 16) < _N_MAX)
            da = _sqrt16(jnp.maximum(d2a, _splat_f(1e-30)))
            db = _sqrt16(jnp.maximum(d2b, _splat_f(1e-30)))
            zv = _splat_f(0.0)
            s1a = s1a + jnp.where(maska, da, zv)
            s1b = s1b + jnp.where(maskb, db, zv)
            s2a = s2a + jnp.where(maska, d2a, zv)
            s2b = s2b + jnp.where(maskb, d2b, zv)
            return (s1a, s1b, s2a, s2b)

        s1a, s1b, s2a, s2b = lax.fori_loop(0, _N_MAX - 1, _pair,
                                           (zero, zero, zero, zero))
        npairs = float(_N_MAX * (_N_MAX - 1) // 2)
        s1v = _splat_f(jnp.sum(s1a) + jnp.sum(s1b))
        s2v = _splat_f(jnp.sum(s2a) + jnp.sum(s2b))
        meanv = s1v / _splat_f(npairs)
        varv = (s2v - _splat_f(npairs) * meanv * meanv) / _splat_f(npairs - 1.0)
        regv = _splat_f(_REG_WEIGHT) * _sqrt16(
            jnp.maximum(varv, _splat_f(1e-30)))
        ostage_v[...] = regv
        pltpu.sync_copy(ostage_v, out_ref)


def _reg_weighted(latent, R_xyz):
    mesh = plsc.VectorSubcoreMesh(core_axis_name="c", subcore_axis_name="s",
                                  num_cores=2, num_subcores=16)
    out = pl.kernel(
        _reg_body,
        out_type=jax.ShapeDtypeStruct((16,), jnp.float32),
        mesh=mesh,
        scratch_types=[
            pltpu.VMEM((_CHUNK,), jnp.float32),
            pltpu.VMEM((3 * _CHUNK,), jnp.float32),
            pltpu.VMEM((_NW * _NSLOT,), jnp.float32),
            pltpu.VMEM((_NW * _NSLOT,), jnp.float32),
            pltpu.VMEM((_NW * _NSLOT,), jnp.float32),
            pltpu.VMEM((_NW * _NSLOT,), jnp.float32),
            pltpu.VMEM((32,), jnp.float32),
            pltpu.VMEM((32,), jnp.float32),
            pltpu.VMEM((32,), jnp.float32),
            pltpu.VMEM((_NSLOT,), jnp.float32),
            pltpu.VMEM((_NSLOT,), jnp.float32),
            pltpu.VMEM((_NSLOT,), jnp.float32),
            pltpu.VMEM((_NSLOT,), jnp.float32),
            pltpu.VMEM((16,), jnp.float32),
            pltpu.VMEM_SHARED((_NW * _NSLOT,), jnp.float32),
            pltpu.VMEM_SHARED((_NW * _NSLOT,), jnp.float32),
            pltpu.VMEM_SHARED((_NW * _NSLOT,), jnp.float32),
            pltpu.VMEM_SHARED((_NW * _NSLOT,), jnp.float32),
        ],
        compiler_params=pltpu.CompilerParams(needs_layout_passes=False),
    )(latent, R_xyz.reshape(-1))
    return out[0]


def kernel(target, pred, latent, R_xyz):
    l1 = _l1_mean(target, pred)
    regw = _reg_weighted(latent, R_xyz)
    total = l1 + regw
    return (total, l1, regw)


# speedup vs baseline: 1.0239x; 1.0239x over previous
import jax
import jax.numpy as jnp
from jax import lax
from jax.experimental import pallas as pl
from jax.experimental.pallas import tpu as pltpu
from jax.experimental.pallas import tpu_sc as plsc

_N_MAX = 20
_REG_WEIGHT = 0.01
_ROWS, _COLS = 128, 32768
_ROW_BLOCK = 16
_NSTEPS = _ROWS // _ROW_BLOCK

_NW = 16
_CHUNK = _COLS // _NW
_NGROUP = _CHUNK // 256
_NSLOT = 32
_BIG = 3.0e38


def _l1_body(t_ref, p_ref, l1_ref):
    step = pl.program_id(0)
    bsum = jnp.sum(jnp.abs(t_ref[...] - p_ref[...]))

    @pl.when(step == 0)
    def _init():
        l1_ref[...] = jnp.reshape(bsum, (1, 1))

    @pl.when(step != 0)
    def _acc():
        l1_ref[...] += jnp.reshape(bsum, (1, 1))

    @pl.when(step == _NSTEPS - 1)
    def _fin():
        l1_ref[...] = l1_ref[...] / float(_ROWS * _COLS)


def _l1_mean(target, pred):
    out = pl.pallas_call(
        _l1_body,
        grid=(_NSTEPS,),
        in_specs=[
            pl.BlockSpec((_ROW_BLOCK, _COLS), lambda i: (i, 0)),
            pl.BlockSpec((_ROW_BLOCK, _COLS), lambda i: (i, 0)),
        ],
        out_specs=pl.BlockSpec((1, 1), lambda i: (0, 0)),
        out_shape=jax.ShapeDtypeStruct((1, 1), jnp.float32),
        compiler_params=pltpu.CompilerParams(
            dimension_semantics=("arbitrary",),
        ),
    )(target, pred)
    return out[0, 0]


def _iota16():
    return lax.iota(jnp.int32, 16)


def _splat_f(x):
    return jnp.full((16,), x, dtype=jnp.float32)


def _splat_i(x):
    return jnp.full((16,), x, dtype=jnp.int32)


def _sqrt16(a):
    ai = plsc.bitcast(a, jnp.int32)
    x = plsc.bitcast((ai >> _splat_i(1)) + _splat_i(0x1FBD1DF5), jnp.float32)
    for _ in range(3):
        x = _splat_f(0.5) * (x + a / x)
    return x


def _argmax_pos(vec, m):
    return jnp.min(jnp.where(vec == m, _iota16(), _splat_i(9999)))


def _reg_body(lat_ref, r_ref, out_ref,
              chunk_v, rchunk_v, mval_v, mx_v, my_v, mz_v,
              xs_v, ys_v, zs_v,
              vstage_v, xstage_v, ystage_v, zstage_v, ostage_v,
              shval, shx, shy, shz):
    cid = lax.axis_index("c")
    sid = lax.axis_index("s")
    lane = _iota16()
    lane0 = lane == 0

    @pl.when(cid == 0)
    def _local():
        base = sid * _CHUNK
        pltpu.sync_copy(lat_ref.at[pl.ds(base, _CHUNK)], chunk_v)
        for c in range(3):
            pltpu.sync_copy(r_ref.at[pl.ds(c, 1), pl.ds(base, _CHUNK)],
                            rchunk_v.at[pl.ds(c, 1)])

        g_list = []
        for g in range(_NGROUP):
            acc = chunk_v[pl.ds(g * 256, 16)]
            for j in range(1, 16):
                acc = jnp.maximum(acc, chunk_v[pl.ds(g * 256 + j * 16, 16)])
            g_list.append(acc)

        def _round(k, carry):
            pv0, pv1, pi0, pi1, *gs = carry
            mall = gs[0]
            for g in range(1, _NGROUP):
                mall = jnp.maximum(mall, gs[g])
            m = jnp.max(mall)
            g_star = jnp.int32(9999)
            for g in range(_NGROUP):
                hit = jnp.max(jnp.where(gs[g] == m, _splat_i(1), _splat_i(0)))
                g_star = jnp.minimum(
                    g_star, jnp.where(hit > 0, jnp.int32(g), jnp.int32(9999)))
            gsel = _splat_f(-_BIG)
            for g in range(_NGROUP):
                gsel = jnp.where(_splat_i(g_star) == g, gs[g], gsel)
            l_star = _argmax_pos(gsel, m)
            colidx = g_star * 256 + lane * 16 + l_star
            col = plsc.load_gather(chunk_v, [colidx])
            r_off = jnp.min(jnp.where(col == m, lane, _splat_i(9999)))
            flat = g_star * 256 + r_off * 16 + l_star
            plsc.store_scatter(chunk_v, [_splat_i(flat)], _splat_f(-_BIG),
                               mask=lane0)
            newg = _splat_f(-_BIG)
            for j in range(16):
                rowv = plsc.load_gather(chunk_v,
                                        [g_star * 256 + j * 16 + lane])
                newg = jnp.maximum(newg, rowv)
            new_gs = [jnp.where(_splat_i(g_star) == g, newg, gs[g])
                      for g in range(_NGROUP)]
            pv0 = jnp.where(lane == k, _splat_f(m), pv0)
            pv1 = jnp.where(lane == k - 16, _splat_f(m), pv1)
            pi0 = jnp.where(lane == k, _splat_i(flat), pi0)
            pi1 = jnp.where(lane == k - 16, _splat_i(flat), pi1)
            return (pv0, pv1, pi0, pi1, *new_gs)

        init = (_splat_f(-_BIG), _splat_f(-_BIG),
                _splat_i(0), _splat_i(0), *g_list)
        pv0, pv1, pi0, pi1, *_ = lax.fori_loop(0, _N_MAX, _round, init)

        vstage_v[pl.ds(0, 16)] = pv0
        vstage_v[pl.ds(16, 16)] = pv1
        z16 = _splat_i(0)
        xstage_v[pl.ds(0, 16)] = plsc.load_gather(rchunk_v, [z16, pi0])
        xstage_v[pl.ds(16, 16)] = plsc.load_gather(rchunk_v, [z16, pi1])
        ystage_v[pl.ds(0, 16)] = plsc.load_gather(rchunk_v, [z16 + 1, pi0])
        ystage_v[pl.ds(16, 16)] = plsc.load_gather(rchunk_v, [z16 + 1, pi1])
        zstage_v[pl.ds(0, 16)] = plsc.load_gather(rchunk_v, [z16 + 2, pi0])
        zstage_v[pl.ds(16, 16)] = plsc.load_gather(rchunk_v, [z16 + 2, pi1])
        pltpu.sync_copy(vstage_v, shval.at[pl.ds(sid * _NSLOT, _NSLOT)])
        pltpu.sync_copy(xstage_v, shx.at[pl.ds(sid * _NSLOT, _NSLOT)])
        pltpu.sync_copy(ystage_v, shy.at[pl.ds(sid * _NSLOT, _NSLOT)])
        pltpu.sync_copy(zstage_v, shz.at[pl.ds(sid * _NSLOT, _NSLOT)])

    plsc.subcore_barrier()

    @pl.when((cid == 0) & (sid == 0))
    def _merge():
        pltpu.sync_copy(shval, mval_v)
        pltpu.sync_copy(shx, mx_v)
        pltpu.sync_copy(shy, my_v)
        pltpu.sync_copy(shz, mz_v)

        nrows = _NW * _NSLOT // 16

        def _round(k, carry):
            wx0, wx1, wy0, wy1, wz0, wz1 = carry
            mall = mval_v[pl.ds(0, 16)]
            for j in range(1, nrows):
                mall = jnp.maximum(mall, mval_v[pl.ds(j * 16, 16)])
            m = jnp.max(mall)
            l_star = _argmax_pos(mall, m)
            r_star = jnp.int32(9999)
            for h in range(nrows // 16):
                colidx = (lane + h * 16) * 16 + l_star
                col = plsc.load_gather(mval_v, [colidx])
                cand = jnp.min(jnp.where(col == m, lane + h * 16,
                                         _splat_i(9999)))
                r_star = jnp.minimum(r_star, cand)
            flat = r_star * 16 + l_star
            plsc.store_scatter(mval_v, [_splat_i(flat)], _splat_f(-_BIG),
                               mask=lane0)
            fl = _splat_i(flat)
            gx = plsc.load_gather(mx_v, [fl])
            gy = plsc.load_gather(my_v, [fl])
            gz = plsc.load_gather(mz_v, [fl])
            wx0 = jnp.where(lane == k, gx, wx0)
            wx1 = jnp.where(lane == k - 16, gx, wx1)
            wy0 = jnp.where(lane == k, gy, wy0)
            wy1 = jnp.where(lane == k - 16, gy, wy1)
            wz0 = jnp.where(lane == k, gz, wz0)
            wz1 = jnp.where(lane == k - 16, gz, wz1)
            return (wx0, wx1, wy0, wy1, wz0, wz1)

        zero = _splat_f(0.0)
        wx0, wx1, wy0, wy1, wz0, wz1 = lax.fori_loop(
            0, _N_MAX, _round, (zero, zero, zero, zero, zero, zero))

        xs_v[pl.ds(0, 16)] = wx0
        xs_v[pl.ds(16, 16)] = wx1
        ys_v[pl.ds(0, 16)] = wy0
        ys_v[pl.ds(16, 16)] = wy1
        zs_v[pl.ds(0, 16)] = wz0
        zs_v[pl.ds(16, 16)] = wz1

        def _pair(i, carry):
            s1a, s1b, s2a, s2b = carry
            xi = plsc.load_gather(xs_v, [_splat_i(i)])
            yi = plsc.load_gather(ys_v, [_splat_i(i)])
            zi = plsc.load_gather(zs_v, [_splat_i(i)])
            dxa = wx0 - xi
            dya = wy0 - yi
            dza = wz0 - zi
            dxb = wx1 - xi
            dyb = wy1 - yi
            dzb = wz1 - zi
            d2a = dxa * dxa + dya * dya + dza * dza
            d2b = dxb * dxb + dyb * dyb + dzb * dzb
            maska = (lane > i) & (lane < _N_MAX)
            maskb = ((lane + 16) > i) & ((lane + 16) < _N_MAX)
            da = _sqrt16(jnp.maximum(d2a, _splat_f(1e-30)))
            db = _sqrt16(jnp.maximum(d2b, _splat_f(1e-30)))
            zv = _splat_f(0.0)
            s1a = s1a + jnp.where(maska, da, zv)
            s1b = s1b + jnp.where(maskb, db, zv)
            s2a = s2a + jnp.where(maska, d2a, zv)
            s2b = s2b + jnp.where(maskb, d2b, zv)
            return (s1a, s1b, s2a, s2b)

        s1a, s1b, s2a, s2b = lax.fori_loop(0, _N_MAX - 1, _pair,
                                           (zero, zero, zero, zero))
        npairs = float(_N_MAX * (_N_MAX - 1) // 2)
        s1v = _splat_f(jnp.sum(s1a) + jnp.sum(s1b))
        s2v = _splat_f(jnp.sum(s2a) + jnp.sum(s2b))
        meanv = s1v / _splat_f(npairs)
        varv = (s2v - _splat_f(npairs) * meanv * meanv) / _splat_f(npairs - 1.0)
        regv = _splat_f(_REG_WEIGHT) * _sqrt16(
            jnp.maximum(varv, _splat_f(1e-30)))
        ostage_v[...] = regv
        pltpu.sync_copy(ostage_v, out_ref)


def _reg_weighted(latent, R_xyz):
    mesh = plsc.VectorSubcoreMesh(core_axis_name="c", subcore_axis_name="s",
                                  num_cores=1, num_subcores=16)
    out = pl.kernel(
        _reg_body,
        out_type=jax.ShapeDtypeStruct((16,), jnp.float32),
        mesh=mesh,
        scratch_types=[
            pltpu.VMEM((_CHUNK,), jnp.float32),
            pltpu.VMEM((3, _CHUNK), jnp.float32),
            pltpu.VMEM((_NW * _NSLOT,), jnp.float32),
            pltpu.VMEM((_NW * _NSLOT,), jnp.float32),
            pltpu.VMEM((_NW * _NSLOT,), jnp.float32),
            pltpu.VMEM((_NW * _NSLOT,), jnp.float32),
            pltpu.VMEM((32,), jnp.float32),
            pltpu.VMEM((32,), jnp.float32),
            pltpu.VMEM((32,), jnp.float32),
            pltpu.VMEM((_NSLOT,), jnp.float32),
            pltpu.VMEM((_NSLOT,), jnp.float32),
            pltpu.VMEM((_NSLOT,), jnp.float32),
            pltpu.VMEM((_NSLOT,), jnp.float32),
            pltpu.VMEM((16,), jnp.float32),
            pltpu.VMEM_SHARED((_NW * _NSLOT,), jnp.float32),
            pltpu.VMEM_SHARED((_NW * _NSLOT,), jnp.float32),
            pltpu.VMEM_SHARED((_NW * _NSLOT,), jnp.float32),
            pltpu.VMEM_SHARED((_NW * _NSLOT,), jnp.float32),
        ],
        compiler_params=pltpu.CompilerParams(needs_layout_passes=False),
    )(latent, R_xyz)
    return out[0]


def kernel(target, pred, latent, R_xyz):
    l1 = _l1_mean(target, pred)
    regw = _reg_weighted(latent, R_xyz)
    total = l1 + regw
    return (total, l1, regw)
